# baseline (speedup 1.0000x reference)
import functools

import jax
import jax.numpy as jnp
import numpy as np
from jax import lax
from jax.experimental import pallas as pl
from jax.experimental.pallas import tpu as pltpu
from jax.experimental.pallas import tpu_sc as plsc

_NC = 2
_NS = 16
_NW = _NC * _NS
_LANES = 16

_BOUND = 10.0
_L = 64
_INV_BIN_W = float(np.float32(1.0) / np.float32(20.0 / _L))
_UNROLL = 8

_LOG2_POLY = (
    -0.025123203173279762,
    0.2700374722480774,
    -1.247962474822998,
    3.24946665763855,
    -5.301709175109863,
    6.089895725250244,
    -3.0346028804779053,
)


def _log2_vec(x):
    i = plsc.bitcast(x, jnp.int32)
    e = (lax.shift_right_logical(i, 23) - 127).astype(jnp.float32)
    m = plsc.bitcast((i & 0x7FFFFF) | 0x3F800000, jnp.float32)
    p = jnp.full((_LANES,), _LOG2_POLY[0], jnp.float32)
    for c in _LOG2_POLY[1:]:
        p = p * m + c
    return e + p


def _quant_body(z_ref, zq_ref, idx_ref):
    v = z_ref[0]
    zc = jnp.minimum(jnp.maximum(v, -_BOUND), _BOUND)
    zq = jnp.round(zc)
    zq_ref[0] = zq
    t = (zq + _BOUND) * _INV_BIN_W
    idx = jnp.minimum(t.astype(jnp.int32), _L - 1)
    r = idx.reshape(1024, 128)
    idx_ref[...] = (r[0:256] | (r[256:512] << 8) | (r[512:768] << 16)
                    | (r[768:1024] << 24))


def _final_sum_body(ps_ref, out_ref):
    out_ref[0, 0] = jnp.sum(ps_ref[...])


def _sc_body(idx_hbm, cdf_hbm, psum_hbm, buf, cdfv, tbl, accv, rows_per_tile):
    cid = lax.axis_index("c")
    sid = lax.axis_index("s")
    wid = sid * _NC + cid
    base = wid * rows_per_tile

    pltpu.sync_copy(cdf_hbm, cdfv)
    pltpu.sync_copy(idx_hbm.at[pl.ds(base, rows_per_tile)], buf)

    nv = _L // _LANES
    lane = lax.iota(jnp.int32, _LANES)
    zero16 = jnp.zeros((_LANES,), jnp.int32)
    rows = [plsc.load_gather(cdfv, [lane + u * _LANES, zero16])
            for u in range(nv)]
    m = rows[0]
    for r in rows[1:]:
        m = jnp.maximum(m, r)
    mx = jnp.max(m)
    exps = [jnp.exp(r - mx) for r in rows]
    s = exps[0]
    for e in exps[1:]:
        s = s + e
    inv = 1.0 / jnp.full((_LANES,), jnp.sum(s), jnp.float32)
    for u in range(nv):
        tbl[pl.ds(u * _LANES, _LANES)] = -_log2_vec(exps[u] * inv + 1e-9)

    zero = jnp.zeros((_LANES,), jnp.float32)

    @plsc.parallel_loop(0, rows_per_tile, step=1, unroll=4,
                        carry=(zero,) * _UNROLL)
    def accs(r, accs_in):
        accs_out = list(accs_in)
        for u in range(8):
            word = buf[r, pl.ds(u * _LANES, _LANES)]
            lanes = (
                word & 0xFF,
                lax.shift_right_logical(word, 8) & 0xFF,
                lax.shift_right_logical(word, 16) & 0xFF,
                lax.shift_right_logical(word, 24),
            )
            for k, idx in enumerate(lanes):
                a = (4 * u + k) % _UNROLL
                accs_out[a] = accs_out[a] + plsc.load_gather(tbl, [idx])
        return tuple(accs_out)

    acc = accs[0]
    for u in range(1, _UNROLL):
        acc = acc + accs[u]
    accv[...] = acc
    pltpu.sync_copy(accv, psum_hbm.at[wid])


@jax.jit
def _entropy_model(z, cdf_params):
    n = z.size

    z_t = jnp.transpose(z, (0, 2, 3, 1))
    z_t = pltpu.with_memory_space_constraint(z_t, pltpu.HBM)
    zq_t, idx2d = pl.pallas_call(
        _quant_body,
        grid=(z_t.shape[0],),
        in_specs=[pl.BlockSpec((1,) + z_t.shape[1:], lambda i: (i, 0, 0, 0))],
        out_specs=[
            pl.BlockSpec((1,) + z_t.shape[1:], lambda i: (i, 0, 0, 0)),
            pl.BlockSpec((n // z_t.shape[0] // 512, 128), lambda i: (i, 0)),
        ],
        out_shape=[
            jax.ShapeDtypeStruct(z_t.shape, jnp.float32),
            jax.ShapeDtypeStruct((n // 512, 128), jnp.int32),
        ],
    )(z_t)
    zq = jnp.transpose(zq_t, (0, 3, 1, 2))

    rows_per_tile = idx2d.shape[0] // _NW
    mesh = plsc.VectorSubcoreMesh(core_axis_name="c", subcore_axis_name="s")
    psums = pl.kernel(
        functools.partial(_sc_body, rows_per_tile=rows_per_tile),
        mesh=mesh,
        out_type=jax.ShapeDtypeStruct((_NW, _LANES), jnp.float32),
        scratch_types=[
            pltpu.VMEM((rows_per_tile, 128), jnp.int32),
            pltpu.VMEM((_L, 128), jnp.float32),
            pltpu.VMEM((_L,), jnp.float32),
            pltpu.VMEM((_LANES,), jnp.float32),
        ],
        compiler_params=pltpu.CompilerParams(needs_layout_passes=False),
    )(pltpu.with_memory_space_constraint(idx2d, pltpu.HBM),
      pltpu.with_memory_space_constraint(cdf_params.T, pltpu.HBM))

    bits_sum = pl.pallas_call(
        _final_sum_body,
        out_shape=jax.ShapeDtypeStruct((1, 1), jnp.float32),
        out_specs=pl.BlockSpec(memory_space=pltpu.SMEM),
    )(pltpu.with_memory_space_constraint(psums, pltpu.HBM))

    return bits_sum[0, 0], zq


def kernel(z, cdf_params, training):
    return _entropy_model(z, cdf_params)


# speedup vs baseline: 1.0872x; 1.0872x over previous
import functools

import jax
import jax.numpy as jnp
import numpy as np
from jax import lax
from jax.experimental import pallas as pl
from jax.experimental.pallas import tpu as pltpu
from jax.experimental.pallas import tpu_sc as plsc

_NC = 2
_NS = 16
_NW = _NC * _NS
_LANES = 16

_BOUND = 10.0
_L = 64
_INV_BIN_W = float(np.float32(1.0) / np.float32(20.0 / _L))
_UNROLL = 8

_LOG2_POLY = (
    -0.025123203173279762,
    0.2700374722480774,
    -1.247962474822998,
    3.24946665763855,
    -5.301709175109863,
    6.089895725250244,
    -3.0346028804779053,
)


def _log2_vec(x):
    i = plsc.bitcast(x, jnp.int32)
    e = (lax.shift_right_logical(i, 23) - 127).astype(jnp.float32)
    m = plsc.bitcast((i & 0x7FFFFF) | 0x3F800000, jnp.float32)
    p = jnp.full((_LANES,), _LOG2_POLY[0], jnp.float32)
    for c in _LOG2_POLY[1:]:
        p = p * m + c
    return e + p


def _quant_body(z_ref, zq_ref, idx_ref):
    v = z_ref[0]
    zc = jnp.minimum(jnp.maximum(v, -_BOUND), _BOUND)
    zq = jnp.round(zc)
    zq_ref[0] = zq
    t = (zq + _BOUND) * _INV_BIN_W
    idx = jnp.minimum(t.astype(jnp.int32), _L - 1)
    r = idx.reshape(1024, 128)
    idx_ref[...] = (r[0:256] | (r[256:512] << 8) | (r[512:768] << 16)
                    | (r[768:1024] << 24))


def _final_sum_body(ps_ref, out_ref):
    out_ref[0, 0] = jnp.sum(ps_ref[...])


def _sc_body(idx_hbm, cdf_hbm, psum_hbm, buf, cdfv, tbl, accv, rows_per_tile):
    cid = lax.axis_index("c")
    sid = lax.axis_index("s")
    wid = sid * _NC + cid
    base = wid * rows_per_tile

    pltpu.sync_copy(cdf_hbm, cdfv)
    pltpu.sync_copy(idx_hbm.at[pl.ds(base, rows_per_tile)], buf)

    nv = _L // _LANES
    lane = lax.iota(jnp.int32, _LANES)
    zero16 = jnp.zeros((_LANES,), jnp.int32)
    rows = [plsc.load_gather(cdfv, [lane + u * _LANES, zero16])
            for u in range(nv)]
    m = rows[0]
    for r in rows[1:]:
        m = jnp.maximum(m, r)
    mx = jnp.max(m)
    exps = [jnp.exp(r - mx) for r in rows]
    s = exps[0]
    for e in exps[1:]:
        s = s + e
    inv = 1.0 / jnp.full((_LANES,), jnp.sum(s), jnp.float32)
    for u in range(nv):
        tbl[pl.ds(u * _LANES, _LANES)] = -_log2_vec(exps[u] * inv + 1e-9)

    zero = jnp.zeros((_LANES,), jnp.float32)

    @plsc.parallel_loop(0, rows_per_tile, step=1, carry=(zero,) * _UNROLL)
    def accs(r, accs_in):
        accs_out = list(accs_in)
        for u in range(8):
            word = buf[r, pl.ds(u * _LANES, _LANES)]
            lanes = (
                word & 0xFF,
                lax.shift_right_logical(word, 8) & 0xFF,
                lax.shift_right_logical(word, 16) & 0xFF,
                lax.shift_right_logical(word, 24),
            )
            for k, idx in enumerate(lanes):
                a = (4 * u + k) % _UNROLL
                accs_out[a] = accs_out[a] + plsc.load_gather(tbl, [idx])
        return tuple(accs_out)

    acc = accs[0]
    for u in range(1, _UNROLL):
        acc = acc + accs[u]
    accv[...] = acc
    pltpu.sync_copy(accv, psum_hbm.at[wid])


@jax.jit
def _entropy_model(z, cdf_params):
    n = z.size

    z_t = jnp.transpose(z, (0, 2, 3, 1))
    z_t = pltpu.with_memory_space_constraint(z_t, pltpu.HBM)
    zq_t, idx2d = pl.pallas_call(
        _quant_body,
        grid=(z_t.shape[0],),
        in_specs=[pl.BlockSpec((1,) + z_t.shape[1:], lambda i: (i, 0, 0, 0))],
        out_specs=[
            pl.BlockSpec((1,) + z_t.shape[1:], lambda i: (i, 0, 0, 0)),
            pl.BlockSpec((n // z_t.shape[0] // 512, 128), lambda i: (i, 0)),
        ],
        out_shape=[
            jax.ShapeDtypeStruct(z_t.shape, jnp.float32),
            jax.ShapeDtypeStruct((n // 512, 128), jnp.int32),
        ],
    )(z_t)
    zq = jnp.transpose(zq_t, (0, 3, 1, 2))

    rows_per_tile = idx2d.shape[0] // _NW
    mesh = plsc.VectorSubcoreMesh(core_axis_name="c", subcore_axis_name="s")
    psums = pl.kernel(
        functools.partial(_sc_body, rows_per_tile=rows_per_tile),
        mesh=mesh,
        out_type=jax.ShapeDtypeStruct((_NW, _LANES), jnp.float32),
        scratch_types=[
            pltpu.VMEM((rows_per_tile, 128), jnp.int32),
            pltpu.VMEM((_L, 128), jnp.float32),
            pltpu.VMEM((_L,), jnp.float32),
            pltpu.VMEM((_LANES,), jnp.float32),
        ],
        compiler_params=pltpu.CompilerParams(needs_layout_passes=False),
    )(pltpu.with_memory_space_constraint(idx2d, pltpu.HBM),
      pltpu.with_memory_space_constraint(cdf_params.T, pltpu.HBM))

    bits_sum = pl.pallas_call(
        _final_sum_body,
        out_shape=jax.ShapeDtypeStruct((1, 1), jnp.float32),
        out_specs=pl.BlockSpec(memory_space=pltpu.SMEM),
    )(pltpu.with_memory_space_constraint(psums, pltpu.HBM))

    return bits_sum[0, 0], zq


def kernel(z, cdf_params, training):
    return _entropy_model(z, cdf_params)


# speedup vs baseline: 1.1396x; 1.0481x over previous
import functools

import jax
import jax.numpy as jnp
import numpy as np
from jax import lax
from jax.experimental import pallas as pl
from jax.experimental.pallas import tpu as pltpu
from jax.experimental.pallas import tpu_sc as plsc

_NC = 2
_NS = 16
_NW = _NC * _NS
_LANES = 16

_BOUND = 10.0
_L = 64
_MAGIC = 12582912.0
_INV_BIN_W = float(np.float32(1.0) / np.float32(20.0 / _L))
_UNROLL = 8

_LOG2_POLY = (
    -0.025123203173279762,
    0.2700374722480774,
    -1.247962474822998,
    3.24946665763855,
    -5.301709175109863,
    6.089895725250244,
    -3.0346028804779053,
)


def _log2_vec(x):
    i = plsc.bitcast(x, jnp.int32)
    e = (lax.shift_right_logical(i, 23) - 127).astype(jnp.float32)
    m = plsc.bitcast((i & 0x7FFFFF) | 0x3F800000, jnp.float32)
    p = jnp.full((_LANES,), _LOG2_POLY[0], jnp.float32)
    for c in _LOG2_POLY[1:]:
        p = p * m + c
    return e + p


def _quant_body(z_ref, zq_ref):
    v = z_ref[0]
    zc = jnp.minimum(jnp.maximum(v, -_BOUND), _BOUND)
    zq_ref[0] = jnp.round(zc)


def _final_sum_body(ps_ref, out_ref):
    out_ref[0, 0] = jnp.sum(ps_ref[...])


def _sc_body(z_hbm, cdf_hbm, psum_hbm, buf, cdfv, tbl, accv, rows_per_tile):
    cid = lax.axis_index("c")
    sid = lax.axis_index("s")
    wid = sid * _NC + cid
    base = wid * rows_per_tile

    pltpu.sync_copy(cdf_hbm, cdfv)
    pltpu.sync_copy(z_hbm.at[pl.ds(base, rows_per_tile)], buf)

    nv = _L // _LANES
    rows = [cdfv[pl.ds(u * _LANES, _LANES)] for u in range(nv)]
    m = rows[0]
    for r in rows[1:]:
        m = jnp.maximum(m, r)
    mx = jnp.max(m)
    exps = [jnp.exp(r - mx) for r in rows]
    s = exps[0]
    for e in exps[1:]:
        s = s + e
    inv = 1.0 / jnp.full((_LANES,), jnp.sum(s), jnp.float32)
    for u in range(nv):
        tbl[pl.ds(u * _LANES, _LANES)] = -_log2_vec(exps[u] * inv + 1e-9)

    zero = jnp.zeros((_LANES,), jnp.float32)

    @plsc.parallel_loop(0, rows_per_tile, step=1, carry=(zero,) * _UNROLL)
    def accs(r, accs_in):
        accs_out = []
        for u in range(_UNROLL):
            v = buf[r, pl.ds(u * _LANES, _LANES)]
            zc = jnp.minimum(jnp.maximum(v, -_BOUND), _BOUND)
            zq = (zc + _MAGIC) - _MAGIC
            t = (zq + _BOUND) * _INV_BIN_W
            idx = jnp.minimum(t.astype(jnp.int32), _L - 1)
            accs_out.append(accs_in[u] + plsc.load_gather(tbl, [idx]))
        return tuple(accs_out)

    acc = accs[0]
    for u in range(1, _UNROLL):
        acc = acc + accs[u]
    accv[...] = acc
    pltpu.sync_copy(accv, psum_hbm.at[wid])


@jax.jit
def _entropy_model(z, cdf_params):
    n = z.size

    z_t = jnp.transpose(z, (0, 2, 3, 1))
    z_lin = z_t.reshape(n // 128, 128)

    rows_per_tile = (n // 128) // _NW
    mesh = plsc.VectorSubcoreMesh(core_axis_name="c", subcore_axis_name="s")
    psums = pl.kernel(
        functools.partial(_sc_body, rows_per_tile=rows_per_tile),
        mesh=mesh,
        out_type=jax.ShapeDtypeStruct((_NW, _LANES), jnp.float32),
        scratch_types=[
            pltpu.VMEM((rows_per_tile, 128), jnp.float32),
            pltpu.VMEM((_L,), jnp.float32),
            pltpu.VMEM((_L,), jnp.float32),
            pltpu.VMEM((_LANES,), jnp.float32),
        ],
        compiler_params=pltpu.CompilerParams(needs_layout_passes=False),
    )(pltpu.with_memory_space_constraint(z_lin, pltpu.HBM), cdf_params[0])

    zq_t = pl.pallas_call(
        _quant_body,
        grid=(z_t.shape[0],),
        in_specs=[pl.BlockSpec((1,) + z_t.shape[1:], lambda i: (i, 0, 0, 0))],
        out_specs=pl.BlockSpec((1,) + z_t.shape[1:], lambda i: (i, 0, 0, 0)),
        out_shape=jax.ShapeDtypeStruct(z_t.shape, jnp.float32),
    )(pltpu.with_memory_space_constraint(z_t, pltpu.HBM))
    zq = jnp.transpose(zq_t, (0, 3, 1, 2))

    bits_sum = pl.pallas_call(
        _final_sum_body,
        out_shape=jax.ShapeDtypeStruct((1, 1), jnp.float32),
        out_specs=pl.BlockSpec(memory_space=pltpu.SMEM),
    )(pltpu.with_memory_space_constraint(psums, pltpu.HBM))

    return bits_sum[0, 0], zq


def kernel(z, cdf_params, training):
    return _entropy_model(z, cdf_params)
